# Initial kernel scaffold; baseline (speedup 1.0000x reference)
import jax
import jax.numpy as jnp
from jax.experimental import pallas as pl


def kernel(x, edge_index, W1, b1, W2, b2):
    raise NotImplementedError("write your pallas kernel here")


# speedup vs baseline: 13.0945x; 13.0945x over previous
import functools

import jax
import jax.numpy as jnp
from jax import lax
from jax.experimental import pallas as pl
from jax.experimental.pallas import tpu as pltpu
from jax.experimental.pallas import tpu_sc as plsc

N_NODES = 10000
N_EDGES = 320000
D = 128

NPAD = 10240
EB = 80
TILES = 32
E_PER_TILE = N_EDGES // TILES
BATCHES = E_PER_TILE // EB
AROWS_PER_TILE = NPAD // 16
ZCHUNK = 128

_mesh = plsc.VectorSubcoreMesh(core_axis_name="c", subcore_axis_name="s")


@functools.partial(
    pl.kernel,
    mesh=_mesh,
    out_type=jax.ShapeDtypeStruct((2, NPAD), jnp.float32),
    scratch_types=[
        pltpu.VMEM_SHARED((NPAD,), jnp.float32),
        pltpu.VMEM((EB,), jnp.int32),
        pltpu.VMEM((EB,), jnp.float32),
        pltpu.VMEM((640,), jnp.float32),
    ],
)
def _deg_sc(dst_hbm, out_hbm, hist, didx, ones, zeros):
    c = lax.axis_index("c")
    s = lax.axis_index("s")

    def fill_ones(i, carry):
        ones[pl.ds(i * 16, 16)] = jnp.full((16,), 1.0, jnp.float32)
        return carry

    lax.fori_loop(0, EB // 16, fill_ones, 0)

    def fill_zeros(i, carry):
        zeros[pl.ds(i * 16, 16)] = jnp.zeros((16,), jnp.float32)
        return carry

    lax.fori_loop(0, 640 // 16, fill_zeros, 0)

    pltpu.sync_copy(zeros, hist.at[pl.ds(s * 640, 640)])
    plsc.subcore_barrier()

    ebase = c * (N_EDGES // 2) + s * E_PER_TILE

    def body(b, carry):
        pltpu.sync_copy(dst_hbm.at[pl.ds(ebase + b * EB, EB)], didx)
        pltpu.sync_copy(ones, hist.at[didx], add=True)
        return carry

    lax.fori_loop(0, BATCHES, body, 0)
    plsc.subcore_barrier()
    pltpu.sync_copy(hist.at[pl.ds(s * 640, 640)],
                    out_hbm.at[c, pl.ds(s * 640, 640)])


@functools.partial(
    pl.kernel,
    mesh=_mesh,
    out_type=jax.ShapeDtypeStruct((2, NPAD, D), jnp.float32),
    scratch_types=[
        pltpu.VMEM_SHARED((NPAD, D), jnp.float32),
        pltpu.VMEM((EB,), jnp.int32),
        pltpu.VMEM((EB,), jnp.int32),
        pltpu.VMEM((EB, D), jnp.float32),
        pltpu.VMEM((ZCHUNK, D), jnp.float32),
        pltpu.SemaphoreType.DMA,
    ],
)
def _agg_sc(xw_hbm, src_hbm, dst_hbm, out_hbm, acc, sidx, didx, rows, zrows, sem):
    c = lax.axis_index("c")
    s = lax.axis_index("s")

    def fill_zrows(i, carry):
        for j in range(D // 16):
            zrows[i, pl.ds(j * 16, 16)] = jnp.zeros((16,), jnp.float32)
        return carry

    lax.fori_loop(0, ZCHUNK, fill_zrows, 0)

    for k in range(AROWS_PER_TILE // ZCHUNK):
        pltpu.sync_copy(zrows, acc.at[pl.ds(s * AROWS_PER_TILE + k * ZCHUNK, ZCHUNK)])
    plsc.subcore_barrier()

    ebase = c * (N_EDGES // 2) + s * E_PER_TILE

    def body(b, carry):
        pltpu.sync_copy(src_hbm.at[pl.ds(ebase + b * EB, EB)], sidx)
        pltpu.sync_copy(dst_hbm.at[pl.ds(ebase + b * EB, EB)], didx)
        pltpu.async_copy(xw_hbm.at[sidx], rows, sem).wait()
        pltpu.sync_copy(rows, acc.at[didx], add=True)
        return carry

    lax.fori_loop(0, BATCHES, body, 0)
    plsc.subcore_barrier()

    for k in range(AROWS_PER_TILE // ZCHUNK):
        r0 = s * AROWS_PER_TILE + k * ZCHUNK
        pltpu.sync_copy(acc.at[pl.ds(r0, ZCHUNK)], out_hbm.at[c, pl.ds(r0, ZCHUNK)])


def _mm1_body(x_ref, w_ref, h0_ref, h1_ref, xwp_ref, dinv_ref):
    deg = h0_ref[...] + h1_ref[...] + 1.0
    dinv = lax.rsqrt(deg)
    xw = jnp.dot(x_ref[...], w_ref[...], preferred_element_type=jnp.float32)
    xwp_ref[...] = xw * dinv
    dinv_ref[...] = dinv


def _mm2_body(xwp_ref, a0_ref, a1_ref, dinv_ref, b_ref, w_ref, out_ref):
    dinv = dinv_ref[...]
    h = (xwp_ref[...] + a0_ref[...] + a1_ref[...]) * dinv + b_ref[...]
    h = jnp.maximum(h, 0.0)
    out_ref[...] = jnp.dot(h, w_ref[...], preferred_element_type=jnp.float32) * dinv


def _fin_body(xwp_ref, a0_ref, a1_ref, dinv_ref, b_ref, out_ref):
    out_ref[...] = ((xwp_ref[...] + a0_ref[...] + a1_ref[...]) * dinv_ref[...]
                    + b_ref[...])


def _mm1(x, w, h0, h1):
    return pl.pallas_call(
        _mm1_body,
        out_shape=(
            jax.ShapeDtypeStruct((N_NODES, D), jnp.float32),
            jax.ShapeDtypeStruct((N_NODES, 1), jnp.float32),
        ),
    )(x, w, h0, h1)


def _mm2(xwp, a0, a1, dinv, b, w):
    return pl.pallas_call(
        _mm2_body,
        out_shape=jax.ShapeDtypeStruct((N_NODES, D), jnp.float32),
    )(xwp, a0, a1, dinv, b, w)


def _fin(xwp, a0, a1, dinv, b):
    return pl.pallas_call(
        _fin_body,
        out_shape=jax.ShapeDtypeStruct((N_NODES, D), jnp.float32),
    )(xwp, a0, a1, dinv, b)


def kernel(x, edge_index, W1, b1, W2, b2):
    ei = edge_index.astype(jnp.int32)
    src = ei[0]
    dst = ei[1]

    hist = _deg_sc(dst)
    h0 = hist[0, :N_NODES].reshape(N_NODES, 1)
    h1 = hist[1, :N_NODES].reshape(N_NODES, 1)

    xw1p, dinv = _mm1(x, W1, h0, h1)
    a = _agg_sc(xw1p, src, dst)
    xw2p = _mm2(xw1p, a[0, :N_NODES], a[1, :N_NODES], dinv,
                b1.reshape(1, D), W2)
    a2 = _agg_sc(xw2p, src, dst)
    return _fin(xw2p, a2[0, :N_NODES], a2[1, :N_NODES], dinv,
                b2.reshape(1, D))
